# Initial kernel scaffold; baseline (speedup 1.0000x reference)
import jax
import jax.numpy as jnp
from jax.experimental import pallas as pl


def kernel(xyz1, xyz2, features1, features2, W1, b1, g1, be1, W2, b2, g2, be2):
    raise NotImplementedError("write your pallas kernel here")


# speedup vs baseline: 15.7028x; 15.7028x over previous
import functools

import jax
import jax.numpy as jnp
from jax import lax
from jax.experimental import pallas as pl
from jax.experimental.pallas import tpu as pltpu
from jax.experimental.pallas import tpu_sc as plsc

B, N1, N2, C1, C2, K = 8, 1024, 4096, 128, 64, 3
H1, H2 = 128, 128
KPAD = 8
TKNN = 512
TM = 512
NT = N2 // TM

NW = 32
PPW = B * N2 // NW
CH = 128
NCH = PPW // CH


def _knn_body(x1_ref, x2_ref, gidx_ref, w_ref):
    b = pl.program_id(0)
    x1 = x1_ref[0]
    x2 = x2_ref[0]
    x1sq = jnp.sum(x1 * x1, axis=1, keepdims=True)
    x2sq = jnp.sum(x2 * x2, axis=1, keepdims=True)
    dot = lax.dot_general(x1, x2, (((1,), (1,)), ((), ())),
                          preferred_element_type=jnp.float32)
    d = (x2sq.T + x1sq) - 2.0 * dot
    iota = lax.broadcasted_iota(jnp.int32, (N1, TKNN), 0)
    inf = jnp.float32(jnp.inf)
    v1 = jnp.min(d, axis=0, keepdims=True)
    i1 = jnp.min(jnp.where(d == v1, iota, N1), axis=0, keepdims=True)
    dm = jnp.where(iota == i1, inf, d)
    v2 = jnp.min(dm, axis=0, keepdims=True)
    i2 = jnp.min(jnp.where(dm == v2, iota, N1), axis=0, keepdims=True)
    dm = jnp.where(iota == i2, inf, dm)
    v3 = jnp.min(dm, axis=0, keepdims=True)
    i3 = jnp.min(jnp.where(dm == v3, iota, N1), axis=0, keepdims=True)
    inv1 = 1.0 / jnp.maximum(v1, 1e-10)
    inv2 = 1.0 / jnp.maximum(v2, 1e-10)
    inv3 = 1.0 / jnp.maximum(v3, 1e-10)
    s = inv1 + inv2 + inv3
    r = lax.broadcasted_iota(jnp.int32, (KPAD, TKNN), 0)
    base = b * N1
    gi = jnp.where(r == 0, i1 + base,
                   jnp.where(r == 1, i2 + base,
                             jnp.where(r == 2, i3 + base, 0)))
    wv = jnp.where(r == 0, inv1 / s,
                   jnp.where(r == 1, inv2 / s,
                             jnp.where(r == 2, inv3 / s, jnp.float32(0.0))))
    gidx_ref[0] = gi
    w_ref[0] = wv


_knn_call = pl.pallas_call(
    _knn_body,
    grid=(B, N2 // TKNN),
    in_specs=[
        pl.BlockSpec((1, N1, 3), lambda b, t: (b, 0, 0)),
        pl.BlockSpec((1, TKNN, 3), lambda b, t: (b, t, 0)),
    ],
    out_specs=[
        pl.BlockSpec((1, KPAD, TKNN), lambda b, t: (b, 0, t)),
        pl.BlockSpec((1, KPAD, TKNN), lambda b, t: (b, 0, t)),
    ],
    out_shape=[
        jax.ShapeDtypeStruct((B, KPAD, N2), jnp.int32),
        jax.ShapeDtypeStruct((B, KPAD, N2), jnp.float32),
    ],
)


def _interp_sc_body(f1p_hbm, gidx_hbm, w_hbm, out_hbm,
                    idx0, idx1, idx2, w0_v, w1_v, w2_v,
                    r0, r1, r2, out_v, sem):
    cid = lax.axis_index("c")
    sid = lax.axis_index("s")
    wid = sid * 2 + cid
    pstart = wid * PPW
    b = pstart // N2
    nbase = pstart - b * N2

    def chunk_body(ch, carry):
        nlo = nbase + ch * CH
        pltpu.sync_copy(gidx_hbm.at[b * KPAD + 0, pl.ds(nlo, CH)], idx0)
        pltpu.sync_copy(gidx_hbm.at[b * KPAD + 1, pl.ds(nlo, CH)], idx1)
        pltpu.sync_copy(gidx_hbm.at[b * KPAD + 2, pl.ds(nlo, CH)], idx2)
        pltpu.sync_copy(w_hbm.at[b * KPAD + 0, pl.ds(nlo, CH)], w0_v)
        pltpu.sync_copy(w_hbm.at[b * KPAD + 1, pl.ds(nlo, CH)], w1_v)
        pltpu.sync_copy(w_hbm.at[b * KPAD + 2, pl.ds(nlo, CH)], w2_v)
        pltpu.async_copy(f1p_hbm.at[idx0], r0, sem).wait()
        pltpu.async_copy(f1p_hbm.at[idx1], r1, sem).wait()
        pltpu.async_copy(f1p_hbm.at[idx2], r2, sem).wait()

        def grp_body(g, c2):
            w0g = w0_v[pl.ds(g * 16, 16)]
            w1g = w1_v[pl.ds(g * 16, 16)]
            w2g = w2_v[pl.ds(g * 16, 16)]
            for j in range(16):
                p = g * 16 + j
                a0 = w0g[j]
                a1 = w1g[j]
                a2 = w2g[j]
                for c in range(C1 // 16):
                    sl = pl.ds(c * 16, 16)
                    out_v[p, sl] = (a0 * r0[p, sl] + a1 * r1[p, sl]
                                    + a2 * r2[p, sl])
            return c2

        lax.fori_loop(0, CH // 16, grp_body, 0)
        pltpu.sync_copy(out_v, out_hbm.at[pl.ds(pstart + ch * CH, CH)])
        return carry

    lax.fori_loop(0, NCH, chunk_body, 0)


_interp_call = pl.kernel(
    _interp_sc_body,
    mesh=plsc.VectorSubcoreMesh(core_axis_name="c", subcore_axis_name="s"),
    out_type=jax.ShapeDtypeStruct((B * N2, C1), jnp.float32),
    scratch_types=[
        pltpu.VMEM((CH,), jnp.int32),
        pltpu.VMEM((CH,), jnp.int32),
        pltpu.VMEM((CH,), jnp.int32),
        pltpu.VMEM((CH,), jnp.float32),
        pltpu.VMEM((CH,), jnp.float32),
        pltpu.VMEM((CH,), jnp.float32),
        pltpu.VMEM((CH, C1), jnp.float32),
        pltpu.VMEM((CH, C1), jnp.float32),
        pltpu.VMEM((CH, C1), jnp.float32),
        pltpu.VMEM((CH, C1), jnp.float32),
        pltpu.SemaphoreType.DMA,
    ],
)


def _mlp1_body(x_ref, f2_ref, w1a_ref, w1b_ref, b1_ref, h1_ref, sum_ref, sq_ref):
    b = pl.program_id(0)
    t = pl.program_id(1)
    x = x_ref[...]
    f2 = f2_ref[0]
    h = (lax.dot_general(w1a_ref[...], x, (((1,), (1,)), ((), ())),
                         preferred_element_type=jnp.float32)
         + lax.dot_general(w1b_ref[...], f2, (((1,), (0,)), ((), ())),
                           preferred_element_type=jnp.float32)
         + b1_ref[...])
    h1_ref[0] = h

    @pl.when(jnp.logical_and(b == 0, t == 0))
    def _():
        sum_ref[...] = jnp.zeros_like(sum_ref)
        sq_ref[...] = jnp.zeros_like(sq_ref)

    sum_ref[...] += jnp.sum(h, axis=1, keepdims=True)
    sq_ref[...] += jnp.sum(h * h, axis=1, keepdims=True)


_mlp1_call = pl.pallas_call(
    _mlp1_body,
    grid=(B, NT),
    in_specs=[
        pl.BlockSpec((TM, C1), lambda b, t: (b * NT + t, 0)),
        pl.BlockSpec((1, C2, TM), lambda b, t: (b, 0, t)),
        pl.BlockSpec((H1, C1), lambda b, t: (0, 0)),
        pl.BlockSpec((H1, C2), lambda b, t: (0, 0)),
        pl.BlockSpec((H1, 1), lambda b, t: (0, 0)),
    ],
    out_specs=[
        pl.BlockSpec((1, H1, TM), lambda b, t: (b, 0, t)),
        pl.BlockSpec((H1, 1), lambda b, t: (0, 0)),
        pl.BlockSpec((H1, 1), lambda b, t: (0, 0)),
    ],
    out_shape=[
        jax.ShapeDtypeStruct((B, H1, N2), jnp.float32),
        jax.ShapeDtypeStruct((H1, 1), jnp.float32),
        jax.ShapeDtypeStruct((H1, 1), jnp.float32),
    ],
)


def _mlp2_body(h1_ref, sc_ref, sh_ref, w2_ref, b2_ref, h2_ref, sum_ref, sq_ref):
    b = pl.program_id(0)
    t = pl.program_id(1)
    x = h1_ref[0]
    y = jnp.maximum(x * sc_ref[...] + sh_ref[...], 0.0)
    h = (lax.dot_general(w2_ref[...], y, (((1,), (0,)), ((), ())),
                         preferred_element_type=jnp.float32)
         + b2_ref[...])
    h2_ref[0] = h

    @pl.when(jnp.logical_and(b == 0, t == 0))
    def _():
        sum_ref[...] = jnp.zeros_like(sum_ref)
        sq_ref[...] = jnp.zeros_like(sq_ref)

    sum_ref[...] += jnp.sum(h, axis=1, keepdims=True)
    sq_ref[...] += jnp.sum(h * h, axis=1, keepdims=True)


_mlp2_call = pl.pallas_call(
    _mlp2_body,
    grid=(B, NT),
    in_specs=[
        pl.BlockSpec((1, H1, TM), lambda b, t: (b, 0, t)),
        pl.BlockSpec((H1, 1), lambda b, t: (0, 0)),
        pl.BlockSpec((H1, 1), lambda b, t: (0, 0)),
        pl.BlockSpec((H2, H1), lambda b, t: (0, 0)),
        pl.BlockSpec((H2, 1), lambda b, t: (0, 0)),
    ],
    out_specs=[
        pl.BlockSpec((1, H2, TM), lambda b, t: (b, 0, t)),
        pl.BlockSpec((H2, 1), lambda b, t: (0, 0)),
        pl.BlockSpec((H2, 1), lambda b, t: (0, 0)),
    ],
    out_shape=[
        jax.ShapeDtypeStruct((B, H2, N2), jnp.float32),
        jax.ShapeDtypeStruct((H2, 1), jnp.float32),
        jax.ShapeDtypeStruct((H2, 1), jnp.float32),
    ],
)


def _bnrelu_body(h2_ref, sc_ref, sh_ref, out_ref):
    x = h2_ref[0]
    out_ref[0] = jnp.maximum(x * sc_ref[...] + sh_ref[...], 0.0)


_bnrelu_call = pl.pallas_call(
    _bnrelu_body,
    grid=(B, NT),
    in_specs=[
        pl.BlockSpec((1, H2, TM), lambda b, t: (b, 0, t)),
        pl.BlockSpec((H2, 1), lambda b, t: (0, 0)),
        pl.BlockSpec((H2, 1), lambda b, t: (0, 0)),
    ],
    out_specs=pl.BlockSpec((1, H2, TM), lambda b, t: (b, 0, t)),
    out_shape=jax.ShapeDtypeStruct((B, H2, N2), jnp.float32),
)


def _bn_coeffs(ssum, ssq, g, be, n):
    mean = ssum[:, 0] / n
    var = ssq[:, 0] / n - mean * mean
    scale = g / jnp.sqrt(var + 1e-5)
    shift = be - mean * scale
    return scale[:, None], shift[:, None]


@jax.jit
def kernel(xyz1, xyz2, features1, features2, W1, b1, g1, be1, W2, b2, g2, be2):
    gidx, w = _knn_call(xyz1, xyz2)
    f1p = jnp.transpose(features1, (0, 2, 1)).reshape(B * N1, C1)
    interp = _interp_call(f1p,
                          gidx.reshape(B * KPAD, N2),
                          w.reshape(B * KPAD, N2))
    h1, s1, q1 = _mlp1_call(interp, features2, W1[:, :C1], W1[:, C1:],
                            b1[:, None])
    sc1, sh1 = _bn_coeffs(s1, q1, g1, be1, B * N2)
    h2, s2, q2 = _mlp2_call(h1, sc1, sh1, W2, b2[:, None])
    sc2, sh2 = _bn_coeffs(s2, q2, g2, be2, B * N2)
    return _bnrelu_call(h2, sc2, sh2)


# speedup vs baseline: 16.4853x; 1.0498x over previous
import functools

import jax
import jax.numpy as jnp
from jax import lax
from jax.experimental import pallas as pl
from jax.experimental.pallas import tpu as pltpu
from jax.experimental.pallas import tpu_sc as plsc

B, N1, N2, C1, C2, K = 8, 1024, 4096, 128, 64, 3
H1, H2 = 128, 128
KPAD = 8
TKNN = 512
TM = 512
NT = N2 // TM

NW = 32
PPW = B * N2 // NW
CH = 128
NCH = PPW // CH


def _knn_body(x1a_ref, x2a_ref, gidx_ref, w_ref):
    b = pl.program_id(0)
    x1a = x1a_ref[0]
    x2a = x2a_ref[0]
    x1sq = x1a[:, 3:4]
    x2sq = (jnp.sum(x2a * x2a, axis=1, keepdims=True)
            - 1.0).T
    dotm2 = lax.dot_general(x1a[:, :3], x2a[:, :3], (((1,), (1,)), ((), ())),
                            preferred_element_type=jnp.float32)
    d = (x2sq + x1sq) + dotm2
    iota = lax.broadcasted_iota(jnp.int32, (N1, TKNN), 0)
    inf = jnp.float32(jnp.inf)
    v1 = jnp.min(d, axis=0, keepdims=True)
    i1 = jnp.min(jnp.where(d == v1, iota, N1), axis=0, keepdims=True)
    dm = jnp.where(iota == i1, inf, d)
    v2 = jnp.min(dm, axis=0, keepdims=True)
    i2 = jnp.min(jnp.where(dm == v2, iota, N1), axis=0, keepdims=True)
    dm = jnp.where(iota == i2, inf, dm)
    v3 = jnp.min(dm, axis=0, keepdims=True)
    i3 = jnp.min(jnp.where(dm == v3, iota, N1), axis=0, keepdims=True)
    inv1 = 1.0 / jnp.maximum(v1, 1e-10)
    inv2 = 1.0 / jnp.maximum(v2, 1e-10)
    inv3 = 1.0 / jnp.maximum(v3, 1e-10)
    s = inv1 + inv2 + inv3
    r = lax.broadcasted_iota(jnp.int32, (KPAD, TKNN), 0)
    base = b * N1
    gi = jnp.where(r == 0, i1 + base,
                   jnp.where(r == 1, i2 + base,
                             jnp.where(r == 2, i3 + base, 0)))
    wv = jnp.where(r == 0, inv1 / s,
                   jnp.where(r == 1, inv2 / s,
                             jnp.where(r == 2, inv3 / s, jnp.float32(0.0))))
    gidx_ref[0] = gi
    w_ref[0] = wv


_knn_call = pl.pallas_call(
    _knn_body,
    grid=(B, N2 // TKNN),
    in_specs=[
        pl.BlockSpec((1, N1, 4), lambda b, t: (b, 0, 0)),
        pl.BlockSpec((1, TKNN, 4), lambda b, t: (b, t, 0)),
    ],
    out_specs=[
        pl.BlockSpec((1, KPAD, TKNN), lambda b, t: (b, 0, t)),
        pl.BlockSpec((1, KPAD, TKNN), lambda b, t: (b, 0, t)),
    ],
    out_shape=[
        jax.ShapeDtypeStruct((B, KPAD, N2), jnp.int32),
        jax.ShapeDtypeStruct((B, KPAD, N2), jnp.float32),
    ],
)


def _interp_sc_body(f1p_hbm, gidx_hbm, w_hbm, out_hbm,
                    idx0, idx1, idx2, w0_v, w1_v, w2_v,
                    r0, r1, r2, out_v, sem, gsem):
    cid = lax.axis_index("c")
    sid = lax.axis_index("s")
    wid = sid * 2 + cid
    pstart = wid * PPW
    b = pstart // N2
    nbase = pstart - b * N2

    def chunk_body(ch, carry):
        off = ch * CH
        nlo = nbase + off
        a0 = pltpu.async_copy(gidx_hbm.at[b * KPAD + 0, pl.ds(nlo, CH)],
                              idx0, sem)
        a1 = pltpu.async_copy(gidx_hbm.at[b * KPAD + 1, pl.ds(nlo, CH)],
                              idx1, sem)
        a2 = pltpu.async_copy(gidx_hbm.at[b * KPAD + 2, pl.ds(nlo, CH)],
                              idx2, sem)
        b0 = pltpu.async_copy(w_hbm.at[b * KPAD + 0, pl.ds(nlo, CH)], w0_v, sem)
        b1 = pltpu.async_copy(w_hbm.at[b * KPAD + 1, pl.ds(nlo, CH)], w1_v, sem)
        b2 = pltpu.async_copy(w_hbm.at[b * KPAD + 2, pl.ds(nlo, CH)], w2_v, sem)
        a0.wait()
        a1.wait()
        a2.wait()
        b0.wait()
        b1.wait()
        b2.wait()
        c0 = pltpu.async_copy(f1p_hbm.at[idx0], r0, gsem)
        c1 = pltpu.async_copy(f1p_hbm.at[idx1], r1, gsem)
        c2 = pltpu.async_copy(f1p_hbm.at[idx2], r2, gsem)
        c0.wait()
        c1.wait()
        c2.wait()

        def grp_body(g, cr):
            w0g = w0_v[pl.ds(g * 16, 16)]
            w1g = w1_v[pl.ds(g * 16, 16)]
            w2g = w2_v[pl.ds(g * 16, 16)]
            for j in range(16):
                p = g * 16 + j
                a0 = w0g[j]
                a1 = w1g[j]
                a2 = w2g[j]
                for c in range(C1 // 16):
                    sl = pl.ds(c * 16, 16)
                    out_v[p, sl] = (a0 * r0[p, sl] + a1 * r1[p, sl]
                                    + a2 * r2[p, sl])
            return cr

        lax.fori_loop(0, CH // 16, grp_body, 0)
        pltpu.sync_copy(out_v, out_hbm.at[pl.ds(pstart + off, CH)])
        return carry

    lax.fori_loop(0, NCH, chunk_body, 0)


_interp_call = pl.kernel(
    _interp_sc_body,
    mesh=plsc.VectorSubcoreMesh(core_axis_name="c", subcore_axis_name="s"),
    out_type=jax.ShapeDtypeStruct((B * N2, C1), jnp.float32),
    scratch_types=[
        pltpu.VMEM((CH,), jnp.int32),
        pltpu.VMEM((CH,), jnp.int32),
        pltpu.VMEM((CH,), jnp.int32),
        pltpu.VMEM((CH,), jnp.float32),
        pltpu.VMEM((CH,), jnp.float32),
        pltpu.VMEM((CH,), jnp.float32),
        pltpu.VMEM((CH, C1), jnp.float32),
        pltpu.VMEM((CH, C1), jnp.float32),
        pltpu.VMEM((CH, C1), jnp.float32),
        pltpu.VMEM((CH, C1), jnp.float32),
        pltpu.SemaphoreType.DMA,
        pltpu.SemaphoreType.DMA,
    ],
)


def _mlp1_body(x_ref, f2_ref, w1a_ref, w1b_ref, b1_ref, h1_ref, sum_ref, sq_ref):
    b = pl.program_id(0)
    t = pl.program_id(1)
    x = x_ref[...]
    f2 = f2_ref[0]
    h = (lax.dot_general(w1a_ref[...], x, (((1,), (1,)), ((), ())),
                         preferred_element_type=jnp.float32)
         + lax.dot_general(w1b_ref[...], f2, (((1,), (0,)), ((), ())),
                           preferred_element_type=jnp.float32)
         + b1_ref[...])
    h1_ref[0] = h

    @pl.when(jnp.logical_and(b == 0, t == 0))
    def _():
        sum_ref[...] = jnp.zeros_like(sum_ref)
        sq_ref[...] = jnp.zeros_like(sq_ref)

    sum_ref[...] += jnp.sum(h, axis=1, keepdims=True)
    sq_ref[...] += jnp.sum(h * h, axis=1, keepdims=True)


_mlp1_call = pl.pallas_call(
    _mlp1_body,
    grid=(B, NT),
    in_specs=[
        pl.BlockSpec((TM, C1), lambda b, t: (b * NT + t, 0)),
        pl.BlockSpec((1, C2, TM), lambda b, t: (b, 0, t)),
        pl.BlockSpec((H1, C1), lambda b, t: (0, 0)),
        pl.BlockSpec((H1, C2), lambda b, t: (0, 0)),
        pl.BlockSpec((H1, 1), lambda b, t: (0, 0)),
    ],
    out_specs=[
        pl.BlockSpec((1, H1, TM), lambda b, t: (b, 0, t)),
        pl.BlockSpec((H1, 1), lambda b, t: (0, 0)),
        pl.BlockSpec((H1, 1), lambda b, t: (0, 0)),
    ],
    out_shape=[
        jax.ShapeDtypeStruct((B, H1, N2), jnp.float32),
        jax.ShapeDtypeStruct((H1, 1), jnp.float32),
        jax.ShapeDtypeStruct((H1, 1), jnp.float32),
    ],
)


def _mlp2_body(h1_ref, s1_ref, q1_ref, g1_ref, be1_ref, w2_ref, b2_ref,
               h2_ref, sum_ref, sq_ref):
    b = pl.program_id(0)
    t = pl.program_id(1)
    n = jnp.float32(B * N2)
    mean = s1_ref[...] / n
    var = q1_ref[...] / n - mean * mean
    scale = g1_ref[...] * lax.rsqrt(var + 1e-5)
    shift = be1_ref[...] - mean * scale
    x = h1_ref[0]
    y = jnp.maximum(x * scale + shift, 0.0)
    h = (lax.dot_general(w2_ref[...], y, (((1,), (0,)), ((), ())),
                         preferred_element_type=jnp.float32)
         + b2_ref[...])
    h2_ref[0] = h

    @pl.when(jnp.logical_and(b == 0, t == 0))
    def _():
        sum_ref[...] = jnp.zeros_like(sum_ref)
        sq_ref[...] = jnp.zeros_like(sq_ref)

    sum_ref[...] += jnp.sum(h, axis=1, keepdims=True)
    sq_ref[...] += jnp.sum(h * h, axis=1, keepdims=True)


_mlp2_call = pl.pallas_call(
    _mlp2_body,
    grid=(B, NT),
    in_specs=[
        pl.BlockSpec((1, H1, TM), lambda b, t: (b, 0, t)),
        pl.BlockSpec((H1, 1), lambda b, t: (0, 0)),
        pl.BlockSpec((H1, 1), lambda b, t: (0, 0)),
        pl.BlockSpec((H1, 1), lambda b, t: (0, 0)),
        pl.BlockSpec((H1, 1), lambda b, t: (0, 0)),
        pl.BlockSpec((H2, H1), lambda b, t: (0, 0)),
        pl.BlockSpec((H2, 1), lambda b, t: (0, 0)),
    ],
    out_specs=[
        pl.BlockSpec((1, H2, TM), lambda b, t: (b, 0, t)),
        pl.BlockSpec((H2, 1), lambda b, t: (0, 0)),
        pl.BlockSpec((H2, 1), lambda b, t: (0, 0)),
    ],
    out_shape=[
        jax.ShapeDtypeStruct((B, H2, N2), jnp.float32),
        jax.ShapeDtypeStruct((H2, 1), jnp.float32),
        jax.ShapeDtypeStruct((H2, 1), jnp.float32),
    ],
)


def _bnrelu_body(h2_ref, s2_ref, q2_ref, g2_ref, be2_ref, out_ref):
    n = jnp.float32(B * N2)
    mean = s2_ref[...] / n
    var = q2_ref[...] / n - mean * mean
    scale = g2_ref[...] * lax.rsqrt(var + 1e-5)
    shift = be2_ref[...] - mean * scale
    x = h2_ref[0]
    out_ref[0] = jnp.maximum(x * scale + shift, 0.0)


_bnrelu_call = pl.pallas_call(
    _bnrelu_body,
    grid=(B, NT),
    in_specs=[
        pl.BlockSpec((1, H2, TM), lambda b, t: (b, 0, t)),
        pl.BlockSpec((H2, 1), lambda b, t: (0, 0)),
        pl.BlockSpec((H2, 1), lambda b, t: (0, 0)),
        pl.BlockSpec((H2, 1), lambda b, t: (0, 0)),
        pl.BlockSpec((H2, 1), lambda b, t: (0, 0)),
    ],
    out_specs=pl.BlockSpec((1, H2, TM), lambda b, t: (b, 0, t)),
    out_shape=jax.ShapeDtypeStruct((B, H2, N2), jnp.float32),
)


@jax.jit
def kernel(xyz1, xyz2, features1, features2, W1, b1, g1, be1, W2, b2, g2, be2):
    x1a = jnp.concatenate(
        [-2.0 * xyz1, jnp.sum(xyz1 * xyz1, axis=2, keepdims=True)], axis=2)
    x2a = jnp.concatenate([xyz2, jnp.ones((B, N2, 1), jnp.float32)], axis=2)
    gidx, w = _knn_call(x1a, x2a)
    f1p = jnp.transpose(features1, (0, 2, 1)).reshape(B * N1, C1)
    interp = _interp_call(f1p,
                          gidx.reshape(B * KPAD, N2),
                          w.reshape(B * KPAD, N2))
    h1, s1, q1 = _mlp1_call(interp, features2, W1[:, :C1], W1[:, C1:],
                            b1[:, None])
    h2, s2, q2 = _mlp2_call(h1, s1, q1, g1[:, None], be1[:, None],
                            W2, b2[:, None])
    return _bnrelu_call(h2, s2, q2, g2[:, None], be2[:, None])


# speedup vs baseline: 18.3184x; 1.1112x over previous
import functools

import jax
import jax.numpy as jnp
from jax import lax
from jax.experimental import pallas as pl
from jax.experimental.pallas import tpu as pltpu
from jax.experimental.pallas import tpu_sc as plsc

B, N1, N2, C1, C2, K = 8, 1024, 4096, 128, 64, 3
H1, H2 = 128, 128
KPAD = 8
TKNN = 1024
TM = 512
NT = N2 // TM

NW = 32
PPW = B * N2 // NW
CH = 64
NCH = PPW // CH
NPAIR = NCH // 2


def _knn_body(x1_ref, x2_ref, f1_ref, gidx_ref, w_ref, f1p_ref):
    b = pl.program_id(0)
    t = pl.program_id(1)

    @pl.when(t == 0)
    def _():
        f1p_ref[0] = f1_ref[0].T
    x1 = x1_ref[0]
    x2 = x2_ref[0]
    x1sq = jnp.sum(x1 * x1, axis=1, keepdims=True)
    x2sq = jnp.sum(x2 * x2, axis=1, keepdims=True).T
    dotm2 = lax.dot_general(-2.0 * x1, x2, (((1,), (1,)), ((), ())),
                            preferred_element_type=jnp.float32)
    d = (x2sq + x1sq) + dotm2
    iota = lax.broadcasted_iota(jnp.int32, (N1, TKNN), 0)
    inf = jnp.float32(jnp.inf)
    v1 = jnp.min(d, axis=0, keepdims=True)
    i1 = jnp.min(jnp.where(d == v1, iota, N1), axis=0, keepdims=True)
    dm = jnp.where(iota == i1, inf, d)
    v2 = jnp.min(dm, axis=0, keepdims=True)
    i2 = jnp.min(jnp.where(dm == v2, iota, N1), axis=0, keepdims=True)
    dm = jnp.where(iota == i2, inf, dm)
    v3 = jnp.min(dm, axis=0, keepdims=True)
    i3 = jnp.min(jnp.where(dm == v3, iota, N1), axis=0, keepdims=True)
    inv1 = 1.0 / jnp.maximum(v1, 1e-10)
    inv2 = 1.0 / jnp.maximum(v2, 1e-10)
    inv3 = 1.0 / jnp.maximum(v3, 1e-10)
    s = inv1 + inv2 + inv3
    r = lax.broadcasted_iota(jnp.int32, (KPAD, TKNN), 0)
    base = b * N1
    gi = jnp.where(r == 0, i1 + base,
                   jnp.where(r == 1, i2 + base,
                             jnp.where(r == 2, i3 + base, 0)))
    wv = jnp.where(r == 0, inv1 / s,
                   jnp.where(r == 1, inv2 / s,
                             jnp.where(r == 2, inv3 / s, jnp.float32(0.0))))
    gidx_ref[0] = gi
    w_ref[0] = wv


_knn_call = pl.pallas_call(
    _knn_body,
    grid=(B, N2 // TKNN),
    in_specs=[
        pl.BlockSpec((1, N1, 3), lambda b, t: (b, 0, 0)),
        pl.BlockSpec((1, TKNN, 3), lambda b, t: (b, t, 0)),
        pl.BlockSpec((1, C1, N1), lambda b, t: (b, 0, 0)),
    ],
    out_specs=[
        pl.BlockSpec((1, KPAD, TKNN), lambda b, t: (b, 0, t)),
        pl.BlockSpec((1, KPAD, TKNN), lambda b, t: (b, 0, t)),
        pl.BlockSpec((1, N1, C1), lambda b, t: (b, 0, 0)),
    ],
    out_shape=[
        jax.ShapeDtypeStruct((B, KPAD, N2), jnp.int32),
        jax.ShapeDtypeStruct((B, KPAD, N2), jnp.float32),
        jax.ShapeDtypeStruct((B, N1, C1), jnp.float32),
    ],
)


def _interp_sc_body(f1p_hbm, gidx_hbm, w_hbm, out_hbm,
                    idx_v, w_v, r0a, r1a, r2a, r0b, r1b, r2b,
                    out_v, sem, gsa, gsb):
    cid = lax.axis_index("c")
    sid = lax.axis_index("s")
    wid = sid * 2 + cid
    pstart = wid * PPW
    b = pstart // N2
    nbase = pstart - b * N2

    ci = pltpu.async_copy(
        gidx_hbm.at[pl.ds(b * KPAD, KPAD), pl.ds(nbase, PPW)], idx_v, sem)
    cw = pltpu.async_copy(
        w_hbm.at[pl.ds(b * KPAD, KPAD), pl.ds(nbase, PPW)], w_v, sem)
    ci.wait()
    cw.wait()

    def fire(off, rr0, rr1, rr2, gs):
        pltpu.async_copy(f1p_hbm.at[idx_v.at[0, pl.ds(off, CH)]], rr0, gs)
        pltpu.async_copy(f1p_hbm.at[idx_v.at[1, pl.ds(off, CH)]], rr1, gs)
        pltpu.async_copy(f1p_hbm.at[idx_v.at[2, pl.ds(off, CH)]], rr2, gs)

    def drain(off, rr0, rr1, rr2, gs):
        pltpu.make_async_copy(f1p_hbm.at[idx_v.at[0, pl.ds(off, CH)]],
                              rr0, gs).wait()
        pltpu.make_async_copy(f1p_hbm.at[idx_v.at[1, pl.ds(off, CH)]],
                              rr1, gs).wait()
        pltpu.make_async_copy(f1p_hbm.at[idx_v.at[2, pl.ds(off, CH)]],
                              rr2, gs).wait()

    def compute(off, rr0, rr1, rr2):
        def grp_body(g, cr):
            w0g = w_v[0, pl.ds(off + g * 16, 16)]
            w1g = w_v[1, pl.ds(off + g * 16, 16)]
            w2g = w_v[2, pl.ds(off + g * 16, 16)]
            for j in range(16):
                p = g * 16 + j
                a0 = w0g[j]
                a1 = w1g[j]
                a2 = w2g[j]
                for c in range(C1 // 16):
                    sl = pl.ds(c * 16, 16)
                    out_v[p, sl] = (a0 * rr0[p, sl] + a1 * rr1[p, sl]
                                    + a2 * rr2[p, sl])
            return cr

        lax.fori_loop(0, CH // 16, grp_body, 0)
        pltpu.sync_copy(out_v, out_hbm.at[pl.ds(pstart + off, CH)])

    fire(0, r0a, r1a, r2a, gsa)

    def pair_body(m, carry):
        offa = (2 * m) * CH
        offb = offa + CH
        fire(offb, r0b, r1b, r2b, gsb)
        drain(offa, r0a, r1a, r2a, gsa)
        compute(offa, r0a, r1a, r2a)

        @pl.when(m < NPAIR - 1)
        def _():
            fire(offa + 2 * CH, r0a, r1a, r2a, gsa)

        drain(offb, r0b, r1b, r2b, gsb)
        compute(offb, r0b, r1b, r2b)
        return carry

    lax.fori_loop(0, NPAIR, pair_body, 0)


_interp_call = pl.kernel(
    _interp_sc_body,
    mesh=plsc.VectorSubcoreMesh(core_axis_name="c", subcore_axis_name="s"),
    out_type=jax.ShapeDtypeStruct((B * N2, C1), jnp.float32),
    scratch_types=[
        pltpu.VMEM((KPAD, PPW), jnp.int32),
        pltpu.VMEM((KPAD, PPW), jnp.float32),
        pltpu.VMEM((CH, C1), jnp.float32),
        pltpu.VMEM((CH, C1), jnp.float32),
        pltpu.VMEM((CH, C1), jnp.float32),
        pltpu.VMEM((CH, C1), jnp.float32),
        pltpu.VMEM((CH, C1), jnp.float32),
        pltpu.VMEM((CH, C1), jnp.float32),
        pltpu.VMEM((CH, C1), jnp.float32),
        pltpu.SemaphoreType.DMA,
        pltpu.SemaphoreType.DMA,
        pltpu.SemaphoreType.DMA,
    ],
)


def _mlp1_body(x_ref, f2_ref, w1_ref, b1_ref, h1_ref, sum_ref, sq_ref):
    b = pl.program_id(0)
    t = pl.program_id(1)
    x = x_ref[...]
    f2 = f2_ref[0]
    w1 = w1_ref[...]
    h = (lax.dot_general(w1[:, :C1], x, (((1,), (1,)), ((), ())),
                         preferred_element_type=jnp.float32)
         + lax.dot_general(w1[:, C1:], f2, (((1,), (0,)), ((), ())),
                           preferred_element_type=jnp.float32)
         + b1_ref[...])
    h1_ref[0] = h

    @pl.when(jnp.logical_and(b == 0, t == 0))
    def _():
        sum_ref[...] = jnp.zeros_like(sum_ref)
        sq_ref[...] = jnp.zeros_like(sq_ref)

    sum_ref[...] += jnp.sum(h, axis=1, keepdims=True)
    sq_ref[...] += jnp.sum(h * h, axis=1, keepdims=True)


_mlp1_call = pl.pallas_call(
    _mlp1_body,
    grid=(B, NT),
    in_specs=[
        pl.BlockSpec((TM, C1), lambda b, t: (b * NT + t, 0)),
        pl.BlockSpec((1, C2, TM), lambda b, t: (b, 0, t)),
        pl.BlockSpec((H1, C1 + C2), lambda b, t: (0, 0)),
        pl.BlockSpec((H1, 1), lambda b, t: (0, 0)),
    ],
    out_specs=[
        pl.BlockSpec((1, H1, TM), lambda b, t: (b, 0, t)),
        pl.BlockSpec((H1, 1), lambda b, t: (0, 0)),
        pl.BlockSpec((H1, 1), lambda b, t: (0, 0)),
    ],
    out_shape=[
        jax.ShapeDtypeStruct((B, H1, N2), jnp.float32),
        jax.ShapeDtypeStruct((H1, 1), jnp.float32),
        jax.ShapeDtypeStruct((H1, 1), jnp.float32),
    ],
)


def _mlp2_body(h1_ref, s1_ref, q1_ref, g1_ref, be1_ref, w2_ref, b2_ref,
               h2_ref, sum_ref, sq_ref):
    b = pl.program_id(0)
    t = pl.program_id(1)
    n = jnp.float32(B * N2)
    mean = s1_ref[...] / n
    var = q1_ref[...] / n - mean * mean
    scale = g1_ref[...] * lax.rsqrt(var + 1e-5)
    shift = be1_ref[...] - mean * scale
    x = h1_ref[0]
    y = jnp.maximum(x * scale + shift, 0.0)
    h = (lax.dot_general(w2_ref[...], y, (((1,), (0,)), ((), ())),
                         preferred_element_type=jnp.float32)
         + b2_ref[...])
    h2_ref[0] = h

    @pl.when(jnp.logical_and(b == 0, t == 0))
    def _():
        sum_ref[...] = jnp.zeros_like(sum_ref)
        sq_ref[...] = jnp.zeros_like(sq_ref)

    sum_ref[...] += jnp.sum(h, axis=1, keepdims=True)
    sq_ref[...] += jnp.sum(h * h, axis=1, keepdims=True)


_mlp2_call = pl.pallas_call(
    _mlp2_body,
    grid=(B, NT),
    in_specs=[
        pl.BlockSpec((1, H1, TM), lambda b, t: (b, 0, t)),
        pl.BlockSpec((H1, 1), lambda b, t: (0, 0)),
        pl.BlockSpec((H1, 1), lambda b, t: (0, 0)),
        pl.BlockSpec((H1, 1), lambda b, t: (0, 0)),
        pl.BlockSpec((H1, 1), lambda b, t: (0, 0)),
        pl.BlockSpec((H2, H1), lambda b, t: (0, 0)),
        pl.BlockSpec((H2, 1), lambda b, t: (0, 0)),
    ],
    out_specs=[
        pl.BlockSpec((1, H2, TM), lambda b, t: (b, 0, t)),
        pl.BlockSpec((H2, 1), lambda b, t: (0, 0)),
        pl.BlockSpec((H2, 1), lambda b, t: (0, 0)),
    ],
    out_shape=[
        jax.ShapeDtypeStruct((B, H2, N2), jnp.float32),
        jax.ShapeDtypeStruct((H2, 1), jnp.float32),
        jax.ShapeDtypeStruct((H2, 1), jnp.float32),
    ],
)


def _bnrelu_body(h2_ref, s2_ref, q2_ref, g2_ref, be2_ref, out_ref):
    n = jnp.float32(B * N2)
    mean = s2_ref[...] / n
    var = q2_ref[...] / n - mean * mean
    scale = g2_ref[...] * lax.rsqrt(var + 1e-5)
    shift = be2_ref[...] - mean * scale
    x = h2_ref[0]
    out_ref[0] = jnp.maximum(x * scale + shift, 0.0)


_bnrelu_call = pl.pallas_call(
    _bnrelu_body,
    grid=(B, NT),
    in_specs=[
        pl.BlockSpec((1, H2, TM), lambda b, t: (b, 0, t)),
        pl.BlockSpec((H2, 1), lambda b, t: (0, 0)),
        pl.BlockSpec((H2, 1), lambda b, t: (0, 0)),
        pl.BlockSpec((H2, 1), lambda b, t: (0, 0)),
        pl.BlockSpec((H2, 1), lambda b, t: (0, 0)),
    ],
    out_specs=pl.BlockSpec((1, H2, TM), lambda b, t: (b, 0, t)),
    out_shape=jax.ShapeDtypeStruct((B, H2, N2), jnp.float32),
)


@jax.jit
def kernel(xyz1, xyz2, features1, features2, W1, b1, g1, be1, W2, b2, g2, be2):
    gidx, w, f1p = _knn_call(xyz1, xyz2, features1)
    interp = _interp_call(f1p.reshape(B * N1, C1),
                          gidx.reshape(B * KPAD, N2),
                          w.reshape(B * KPAD, N2))
    h1, s1, q1 = _mlp1_call(interp, features2, W1, b1[:, None])
    h2, s2, q2 = _mlp2_call(h1, s1, q1, g1[:, None], be1[:, None],
                            W2, b2[:, None])
    return _bnrelu_call(h2, s2, q2, g2[:, None], be2[:, None])
